# Initial kernel scaffold; baseline (speedup 1.0000x reference)
import jax
import jax.numpy as jnp
from jax.experimental import pallas as pl


def kernel(graph_edges, node_feat, edge_feat, node_weight, edge_index, atom_feat, gw_W1, gw_b1, gw_W2, gw_b2, out_W1, out_b1, out_W2, out_b2, lin_W, lin_b):
    raise NotImplementedError("write your pallas kernel here")


# speedup vs baseline: 2.3452x; 2.3452x over previous
import functools

import jax
import jax.numpy as jnp
from jax import lax
from jax.experimental import pallas as pl
from jax.experimental.pallas import tpu as pltpu
from jax.experimental.pallas import tpu_sc as plsc

N_B = 10000
N_E = 160000
N_A = 5000
NODE_DIM = 128
EDGE_DIM = 128
ATOM_DIM = 64
HID = 64

NC = 2
NS = 16
NWK = NC * NS
E_PAD = 163840
EPW = E_PAD // NWK
GCHUNK = 128
GITER = EPW // GCHUNK
SPW = N_E // NWK
SCHUNK = 40
SITER = SPW // SCHUNK
NB_PAD = 10240
RPT = NB_PAD // NS
ZROWS = 128


def _node_tables_body(nf_ref, wsrc_ref, wdst_ref, ts_ref, td_ref):
    x = nf_ref[...]
    ts_ref[...] = jnp.dot(x, wsrc_ref[...], preferred_element_type=jnp.float32)
    td_ref[...] = jnp.dot(x, wdst_ref[...], preferred_element_type=jnp.float32)


def _vert_table_body(af_ref, wv_ref, tv_ref):
    tv_ref[...] = jnp.dot(af_ref[...], wv_ref[...], preferred_element_type=jnp.float32)


def _edge_proj_body(ef_ref, we_ref, be_ref, e_ref):
    e_ref[...] = (jnp.dot(ef_ref[...], we_ref[...], preferred_element_type=jnp.float32)
                  + be_ref[...])


def _edge_mlp_body(g_ref, e_ref, nw_ref, wd_ref, bd_ref, m_ref):
    pre = g_ref[...] + e_ref[...]
    h1 = pre * jax.nn.sigmoid(pre)
    z = jnp.dot(h1, wd_ref[...], preferred_element_type=jnp.float32) + bd_ref[...]
    gate = jax.nn.sigmoid(z[:, :EDGE_DIM])
    zo = z[:, EDGE_DIM:]
    outp = zo * jax.nn.sigmoid(zo)
    m_ref[...] = outp * gate * nw_ref[...]


def _final_body(p_ref, nw_ref, nf_ref, wl_ref, bl_ref, o_ref):
    h = (p_ref[0] + p_ref[1]) * nw_ref[...]
    o_ref[...] = (nf_ref[...]
                  + jnp.dot(h, wl_ref[...], preferred_element_type=jnp.float32)
                  + bl_ref[...])


_MESH = plsc.VectorSubcoreMesh(core_axis_name="c", subcore_axis_name="s",
                               num_cores=NC, num_subcores=NS)


@functools.partial(
    pl.kernel,
    out_type=(jax.ShapeDtypeStruct((E_PAD, 128), jnp.float32),
              jax.ShapeDtypeStruct((E_PAD, 128), jnp.float32)),
    mesh=_MESH,
    scratch_types=[
        pltpu.VMEM((GCHUNK,), jnp.int32),
        pltpu.VMEM((GCHUNK,), jnp.int32),
        pltpu.VMEM((GCHUNK,), jnp.int32),
        pltpu.VMEM((GCHUNK, 128), jnp.float32),
        pltpu.VMEM((GCHUNK, 128), jnp.float32),
        pltpu.VMEM((GCHUNK, 128), jnp.float32),
        pltpu.VMEM((GCHUNK, 128), jnp.float32),
        pltpu.VMEM((GCHUNK, 128), jnp.float32),
        pltpu.SemaphoreType.DMA,
    ],
)
def _sc_gather(src_h, dst_h, vid_h, tsrc_h, tdst_h, tvert_h, nw_h,
               g_out, nwg_out,
               sidx, didx, vidx, srows, drows, vrows, nwrows, gbuf, sem):
    c = lax.axis_index("c")
    s = lax.axis_index("s")
    base = (c * NS + s) * EPW

    def step(i, carry):
        cb = base + i * GCHUNK
        pltpu.sync_copy(src_h.at[pl.ds(cb, GCHUNK)], sidx)
        pltpu.sync_copy(dst_h.at[pl.ds(cb, GCHUNK)], didx)
        pltpu.sync_copy(vid_h.at[pl.ds(cb, GCHUNK)], vidx)
        cp1 = pltpu.async_copy(tsrc_h.at[sidx], srows, sem)
        cp2 = pltpu.async_copy(tdst_h.at[didx], drows, sem)
        cp3 = pltpu.async_copy(tvert_h.at[vidx], vrows, sem)
        cp4 = pltpu.async_copy(nw_h.at[sidx], nwrows, sem)
        cp1.wait()
        cp2.wait()
        cp3.wait()
        cp4.wait()

        def edge_body(e, cin):
            for k in range(8):
                sl = pl.ds(k * 16, 16)
                gbuf[e, sl] = srows[e, sl] + drows[e, sl] + vrows[e, sl]
            return cin

        lax.fori_loop(0, GCHUNK, edge_body, 0)
        pltpu.sync_copy(gbuf, g_out.at[pl.ds(cb, GCHUNK)])
        pltpu.sync_copy(nwrows, nwg_out.at[pl.ds(cb, GCHUNK)])
        return carry

    lax.fori_loop(0, GITER, step, 0)


@functools.partial(
    pl.kernel,
    out_type=jax.ShapeDtypeStruct((NC, NB_PAD, 128), jnp.float32),
    mesh=_MESH,
    scratch_types=[
        pltpu.VMEM((SCHUNK,), jnp.int32),
        pltpu.VMEM((SCHUNK, 128), jnp.float32),
        pltpu.VMEM((ZROWS, 128), jnp.float32),
        pltpu.VMEM_SHARED((NB_PAD, 128), jnp.float32),
    ],
)
def _sc_scatter(m_h, dst_h, part_out, didx, mrows, zbuf, acc):
    c = lax.axis_index("c")
    s = lax.axis_index("s")

    def zrow(e, carry):
        for k in range(8):
            zbuf[e, pl.ds(k * 16, 16)] = jnp.zeros((16,), jnp.float32)
        return carry

    lax.fori_loop(0, ZROWS, zrow, 0)
    for j in range(RPT // ZROWS):
        pltpu.sync_copy(zbuf, acc.at[pl.ds(s * RPT + j * ZROWS, ZROWS)])
    plsc.subcore_barrier()

    base = (c * NS + s) * SPW

    def step(i, carry):
        cb = base + i * SCHUNK
        pltpu.sync_copy(dst_h.at[pl.ds(cb, SCHUNK)], didx)
        pltpu.sync_copy(m_h.at[pl.ds(cb, SCHUNK)], mrows)
        pltpu.sync_copy(mrows, acc.at[didx], add=True)
        return carry

    lax.fori_loop(0, SITER, step, 0)
    plsc.subcore_barrier()
    pltpu.sync_copy(acc.at[pl.ds(s * RPT, RPT)], part_out.at[c, pl.ds(s * RPT, RPT)])


def kernel(graph_edges, node_feat, edge_feat, node_weight, edge_index, atom_feat,
           gw_W1, gw_b1, gw_W2, gw_b2, out_W1, out_b1, out_W2, out_b2, lin_W, lin_b):
    f32 = jnp.float32
    src = graph_edges[0].astype(jnp.int32)
    dst = graph_edges[1].astype(jnp.int32)
    vid = edge_index[:, 1].astype(jnp.int32)
    pad = E_PAD - N_E
    src_p = jnp.pad(src, (0, pad))
    dst_p = jnp.pad(dst, (0, pad))
    vid_p = jnp.pad(vid, (0, pad))

    w_src = jnp.concatenate([gw_W1[0:128], out_W1[0:128]], axis=1)
    w_dst = jnp.concatenate([gw_W1[128:256], out_W1[128:256]], axis=1)
    w_edge = jnp.concatenate([gw_W1[256:384], out_W1[256:384]], axis=1)
    w_vert = jnp.concatenate([gw_W1[384:448], out_W1[384:448]], axis=1)
    b_edge = jnp.concatenate([gw_b1, out_b1]).reshape(1, 128)
    w_diag = jnp.zeros((128, 256), f32)
    w_diag = w_diag.at[0:64, 0:128].set(gw_W2).at[64:128, 128:256].set(out_W2)
    b_diag = jnp.concatenate([gw_b2, out_b2]).reshape(1, 256)
    b_lin = lin_b.reshape(1, 128)

    nb_blk = 2000
    t_src, t_dst = pl.pallas_call(
        _node_tables_body,
        grid=(N_B // nb_blk,),
        in_specs=[pl.BlockSpec((nb_blk, 128), lambda i: (i, 0)),
                  pl.BlockSpec((128, 128), lambda i: (0, 0)),
                  pl.BlockSpec((128, 128), lambda i: (0, 0))],
        out_specs=[pl.BlockSpec((nb_blk, 128), lambda i: (i, 0)),
                   pl.BlockSpec((nb_blk, 128), lambda i: (i, 0))],
        out_shape=[jax.ShapeDtypeStruct((N_B, 128), f32),
                   jax.ShapeDtypeStruct((N_B, 128), f32)],
    )(node_feat, w_src, w_dst)

    t_vert = pl.pallas_call(
        _vert_table_body,
        grid=(1,),
        in_specs=[pl.BlockSpec((N_A, ATOM_DIM), lambda i: (0, 0)),
                  pl.BlockSpec((ATOM_DIM, 128), lambda i: (0, 0))],
        out_specs=pl.BlockSpec((N_A, 128), lambda i: (0, 0)),
        out_shape=jax.ShapeDtypeStruct((N_A, 128), f32),
    )(atom_feat, w_vert)

    ne_blk = 2000
    e_proj = pl.pallas_call(
        _edge_proj_body,
        grid=(N_E // ne_blk,),
        in_specs=[pl.BlockSpec((ne_blk, 128), lambda i: (i, 0)),
                  pl.BlockSpec((128, 128), lambda i: (0, 0)),
                  pl.BlockSpec((1, 128), lambda i: (0, 0))],
        out_specs=pl.BlockSpec((ne_blk, 128), lambda i: (i, 0)),
        out_shape=jax.ShapeDtypeStruct((N_E, 128), f32),
    )(edge_feat, w_edge, b_edge)

    g_sum, nw_src = _sc_gather(src_p, dst_p, vid_p, t_src, t_dst, t_vert, node_weight)

    m = pl.pallas_call(
        _edge_mlp_body,
        grid=(N_E // ne_blk,),
        in_specs=[pl.BlockSpec((ne_blk, 128), lambda i: (i, 0)),
                  pl.BlockSpec((ne_blk, 128), lambda i: (i, 0)),
                  pl.BlockSpec((ne_blk, 128), lambda i: (i, 0)),
                  pl.BlockSpec((128, 256), lambda i: (0, 0)),
                  pl.BlockSpec((1, 256), lambda i: (0, 0))],
        out_specs=pl.BlockSpec((ne_blk, 128), lambda i: (i, 0)),
        out_shape=jax.ShapeDtypeStruct((N_E, 128), f32),
    )(g_sum, e_proj, nw_src, w_diag, b_diag)

    partials = _sc_scatter(m, dst)

    out = pl.pallas_call(
        _final_body,
        grid=(N_B // nb_blk,),
        in_specs=[pl.BlockSpec((NC, nb_blk, 128), lambda i: (0, i, 0)),
                  pl.BlockSpec((nb_blk, 128), lambda i: (i, 0)),
                  pl.BlockSpec((nb_blk, 128), lambda i: (i, 0)),
                  pl.BlockSpec((128, 128), lambda i: (0, 0)),
                  pl.BlockSpec((1, 128), lambda i: (0, 0))],
        out_specs=pl.BlockSpec((nb_blk, 128), lambda i: (i, 0)),
        out_shape=jax.ShapeDtypeStruct((N_B, 128), f32),
    )(partials, node_weight, node_feat, lin_W, b_lin)

    return out


# speedup vs baseline: 2.5101x; 1.0703x over previous
import functools

import jax
import jax.numpy as jnp
from jax import lax
from jax.experimental import pallas as pl
from jax.experimental.pallas import tpu as pltpu
from jax.experimental.pallas import tpu_sc as plsc

N_B = 10000
N_E = 160000
N_A = 5000
NODE_DIM = 128
EDGE_DIM = 128
ATOM_DIM = 64
HID = 64

NC = 2
NS = 16
NWK = NC * NS
E_PAD = 163840
EPW = E_PAD // NWK
GCHUNK = 128
GITER = EPW // GCHUNK
SPW = N_E // NWK
SCHUNK = 128
SITER = SPW // SCHUNK
STAIL = SPW - SITER * SCHUNK
NB_PAD = 10240
RPT = NB_PAD // NS
ZROWS = 128


def _node_tables_body(nf_ref, wsrc_ref, wdst_ref, ts_ref, td_ref):
    x = nf_ref[...]
    ts_ref[...] = jnp.dot(x, wsrc_ref[...], preferred_element_type=jnp.float32)
    td_ref[...] = jnp.dot(x, wdst_ref[...], preferred_element_type=jnp.float32)


def _vert_table_body(af_ref, wv_ref, tv_ref):
    tv_ref[...] = jnp.dot(af_ref[...], wv_ref[...], preferred_element_type=jnp.float32)


def _edge_mlp_body(g_ref, ef_ref, nw_ref, we_ref, be_ref, wd_ref, bd_ref, m_ref):
    eproj = (jnp.dot(ef_ref[...], we_ref[...], preferred_element_type=jnp.float32)
             + be_ref[...])
    pre = g_ref[...] + eproj
    h1 = pre * jax.nn.sigmoid(pre)
    z = jnp.dot(h1, wd_ref[...], preferred_element_type=jnp.float32) + bd_ref[...]
    gate = jax.nn.sigmoid(z[:, :EDGE_DIM])
    zo = z[:, EDGE_DIM:]
    outp = zo * jax.nn.sigmoid(zo)
    m_ref[...] = outp * gate * nw_ref[...]


def _final_body(p_ref, nw_ref, nf_ref, wl_ref, bl_ref, o_ref):
    h = (p_ref[0] + p_ref[1]) * nw_ref[...]
    o_ref[...] = (nf_ref[...]
                  + jnp.dot(h, wl_ref[...], preferred_element_type=jnp.float32)
                  + bl_ref[...])


_MESH = plsc.VectorSubcoreMesh(core_axis_name="c", subcore_axis_name="s",
                               num_cores=NC, num_subcores=NS)


@functools.partial(
    pl.kernel,
    out_type=(jax.ShapeDtypeStruct((E_PAD, 128), jnp.float32),
              jax.ShapeDtypeStruct((E_PAD, 128), jnp.float32)),
    mesh=_MESH,
    scratch_types=[
        pltpu.VMEM((GCHUNK,), jnp.int32),
        pltpu.VMEM((GCHUNK,), jnp.int32),
        pltpu.VMEM((GCHUNK,), jnp.int32),
        pltpu.VMEM((GCHUNK, 128), jnp.float32),
        pltpu.VMEM((GCHUNK, 128), jnp.float32),
        pltpu.VMEM((GCHUNK, 128), jnp.float32),
        pltpu.VMEM((GCHUNK, 128), jnp.float32),
        pltpu.VMEM((GCHUNK, 128), jnp.float32),
        pltpu.SemaphoreType.DMA,
    ],
)
def _sc_gather(src_h, dst_h, vid_h, tsrc_h, tdst_h, tvert_h, nw_h,
               g_out, nwg_out,
               sidx, didx, vidx, srows, drows, vrows, nwrows, gbuf, sem):
    c = lax.axis_index("c")
    s = lax.axis_index("s")
    base = (c * NS + s) * EPW

    def step(i, carry):
        cb = base + i * GCHUNK
        pltpu.sync_copy(src_h.at[pl.ds(cb, GCHUNK)], sidx)
        pltpu.sync_copy(dst_h.at[pl.ds(cb, GCHUNK)], didx)
        pltpu.sync_copy(vid_h.at[pl.ds(cb, GCHUNK)], vidx)
        cp1 = pltpu.async_copy(tsrc_h.at[sidx], srows, sem)
        cp2 = pltpu.async_copy(tdst_h.at[didx], drows, sem)
        cp3 = pltpu.async_copy(tvert_h.at[vidx], vrows, sem)
        cp4 = pltpu.async_copy(nw_h.at[sidx], nwrows, sem)
        cp1.wait()
        cp2.wait()
        cp3.wait()
        cp4.wait()

        def edge_body(e, cin):
            for k in range(8):
                sl = pl.ds(k * 16, 16)
                gbuf[e, sl] = srows[e, sl] + drows[e, sl] + vrows[e, sl]
            return cin

        lax.fori_loop(0, GCHUNK, edge_body, 0)
        pltpu.sync_copy(gbuf, g_out.at[pl.ds(cb, GCHUNK)])
        pltpu.sync_copy(nwrows, nwg_out.at[pl.ds(cb, GCHUNK)])
        return carry

    lax.fori_loop(0, GITER, step, 0)


@functools.partial(
    pl.kernel,
    out_type=jax.ShapeDtypeStruct((NC, NB_PAD, 128), jnp.float32),
    mesh=_MESH,
    scratch_types=[
        pltpu.VMEM((SCHUNK,), jnp.int32),
        pltpu.VMEM((SCHUNK, 128), jnp.float32),
        pltpu.VMEM((STAIL,), jnp.int32),
        pltpu.VMEM((STAIL, 128), jnp.float32),
        pltpu.VMEM((ZROWS, 128), jnp.float32),
        pltpu.VMEM_SHARED((NB_PAD, 128), jnp.float32),
    ],
)
def _sc_scatter(m_h, dst_h, part_out, didx, mrows, tidx, trows, zbuf, acc):
    c = lax.axis_index("c")
    s = lax.axis_index("s")

    def zrow(e, carry):
        for k in range(8):
            zbuf[e, pl.ds(k * 16, 16)] = jnp.zeros((16,), jnp.float32)
        return carry

    lax.fori_loop(0, ZROWS, zrow, 0)
    for j in range(RPT // ZROWS):
        pltpu.sync_copy(zbuf, acc.at[pl.ds(s * RPT + j * ZROWS, ZROWS)])
    plsc.subcore_barrier()

    base = (c * NS + s) * SPW

    def step(i, carry):
        cb = base + i * SCHUNK
        pltpu.sync_copy(dst_h.at[pl.ds(cb, SCHUNK)], didx)
        pltpu.sync_copy(m_h.at[pl.ds(cb, SCHUNK)], mrows)
        pltpu.sync_copy(mrows, acc.at[didx], add=True)
        return carry

    lax.fori_loop(0, SITER, step, 0)
    tb = base + SITER * SCHUNK
    pltpu.sync_copy(dst_h.at[pl.ds(tb, STAIL)], tidx)
    pltpu.sync_copy(m_h.at[pl.ds(tb, STAIL)], trows)
    pltpu.sync_copy(trows, acc.at[tidx], add=True)
    plsc.subcore_barrier()
    pltpu.sync_copy(acc.at[pl.ds(s * RPT, RPT)], part_out.at[c, pl.ds(s * RPT, RPT)])


def kernel(graph_edges, node_feat, edge_feat, node_weight, edge_index, atom_feat,
           gw_W1, gw_b1, gw_W2, gw_b2, out_W1, out_b1, out_W2, out_b2, lin_W, lin_b):
    f32 = jnp.float32
    src = graph_edges[0].astype(jnp.int32)
    dst = graph_edges[1].astype(jnp.int32)
    vid = edge_index[:, 1].astype(jnp.int32)
    pad = E_PAD - N_E
    src_p = jnp.pad(src, (0, pad))
    dst_p = jnp.pad(dst, (0, pad))
    vid_p = jnp.pad(vid, (0, pad))

    w_src = jnp.concatenate([gw_W1[0:128], out_W1[0:128]], axis=1)
    w_dst = jnp.concatenate([gw_W1[128:256], out_W1[128:256]], axis=1)
    w_edge = jnp.concatenate([gw_W1[256:384], out_W1[256:384]], axis=1)
    w_vert = jnp.concatenate([gw_W1[384:448], out_W1[384:448]], axis=1)
    b_edge = jnp.concatenate([gw_b1, out_b1]).reshape(1, 128)
    w_diag = jnp.zeros((128, 256), f32)
    w_diag = w_diag.at[0:64, 0:128].set(gw_W2).at[64:128, 128:256].set(out_W2)
    b_diag = jnp.concatenate([gw_b2, out_b2]).reshape(1, 256)
    b_lin = lin_b.reshape(1, 128)

    nb_blk = 2000
    t_src, t_dst = pl.pallas_call(
        _node_tables_body,
        grid=(N_B // nb_blk,),
        in_specs=[pl.BlockSpec((nb_blk, 128), lambda i: (i, 0)),
                  pl.BlockSpec((128, 128), lambda i: (0, 0)),
                  pl.BlockSpec((128, 128), lambda i: (0, 0))],
        out_specs=[pl.BlockSpec((nb_blk, 128), lambda i: (i, 0)),
                   pl.BlockSpec((nb_blk, 128), lambda i: (i, 0))],
        out_shape=[jax.ShapeDtypeStruct((N_B, 128), f32),
                   jax.ShapeDtypeStruct((N_B, 128), f32)],
    )(node_feat, w_src, w_dst)

    t_vert = pl.pallas_call(
        _vert_table_body,
        grid=(1,),
        in_specs=[pl.BlockSpec((N_A, ATOM_DIM), lambda i: (0, 0)),
                  pl.BlockSpec((ATOM_DIM, 128), lambda i: (0, 0))],
        out_specs=pl.BlockSpec((N_A, 128), lambda i: (0, 0)),
        out_shape=jax.ShapeDtypeStruct((N_A, 128), f32),
    )(atom_feat, w_vert)

    ne_blk = 2000
    g_sum, nw_src = _sc_gather(src_p, dst_p, vid_p, t_src, t_dst, t_vert, node_weight)

    m = pl.pallas_call(
        _edge_mlp_body,
        grid=(N_E // ne_blk,),
        in_specs=[pl.BlockSpec((ne_blk, 128), lambda i: (i, 0)),
                  pl.BlockSpec((ne_blk, 128), lambda i: (i, 0)),
                  pl.BlockSpec((ne_blk, 128), lambda i: (i, 0)),
                  pl.BlockSpec((128, 128), lambda i: (0, 0)),
                  pl.BlockSpec((1, 128), lambda i: (0, 0)),
                  pl.BlockSpec((128, 256), lambda i: (0, 0)),
                  pl.BlockSpec((1, 256), lambda i: (0, 0))],
        out_specs=pl.BlockSpec((ne_blk, 128), lambda i: (i, 0)),
        out_shape=jax.ShapeDtypeStruct((N_E, 128), f32),
    )(g_sum, edge_feat, nw_src, w_edge, b_edge, w_diag, b_diag)

    partials = _sc_scatter(m, dst)

    out = pl.pallas_call(
        _final_body,
        grid=(N_B // nb_blk,),
        in_specs=[pl.BlockSpec((NC, nb_blk, 128), lambda i: (0, i, 0)),
                  pl.BlockSpec((nb_blk, 128), lambda i: (i, 0)),
                  pl.BlockSpec((nb_blk, 128), lambda i: (i, 0)),
                  pl.BlockSpec((128, 128), lambda i: (0, 0)),
                  pl.BlockSpec((1, 128), lambda i: (0, 0))],
        out_specs=pl.BlockSpec((nb_blk, 128), lambda i: (i, 0)),
        out_shape=jax.ShapeDtypeStruct((N_B, 128), f32),
    )(partials, node_weight, node_feat, lin_W, b_lin)

    return out


# speedup vs baseline: 3.1466x; 1.2536x over previous
import functools

import jax
import jax.numpy as jnp
from jax import lax
from jax.experimental import pallas as pl
from jax.experimental.pallas import tpu as pltpu
from jax.experimental.pallas import tpu_sc as plsc

N_B = 10000
N_E = 160000
N_A = 5000
NODE_DIM = 128
EDGE_DIM = 128
ATOM_DIM = 64
HID = 64

NC = 2
NS = 16
NWK = NC * NS
E_PAD = 163840
EPW = E_PAD // NWK
GCHUNK = 64
GPAIRS = EPW // (2 * GCHUNK)
IDX_PAD = E_PAD + GCHUNK
SPW = N_E // NWK
SCHUNK = 128
SITER = SPW // SCHUNK
STAIL = SPW - SITER * SCHUNK
SPAIRS = (SITER - 1) // 2
NB_PAD = 10240
RPT = NB_PAD // NS
ZROWS = 64


def _node_tables_body(nf_ref, nw_ref, wsrc_ref, wdst_ref, ts_ref, td_ref):
    x = nf_ref[...]
    ts_ref[:, 0:128] = jnp.dot(x, wsrc_ref[...], preferred_element_type=jnp.float32)
    ts_ref[:, 128:256] = nw_ref[...]
    td_ref[...] = jnp.dot(x, wdst_ref[...], preferred_element_type=jnp.float32)


def _vert_table_body(af_ref, wv_ref, tv_ref):
    tv_ref[...] = jnp.dot(af_ref[...], wv_ref[...], preferred_element_type=jnp.float32)


def _edge_mlp_body(gnw_ref, ef_ref, we_ref, be_ref, wd_ref, bd_ref, m_ref):
    eproj = (jnp.dot(ef_ref[...], we_ref[...], preferred_element_type=jnp.float32)
             + be_ref[...])
    pre = gnw_ref[:, 0:128] + eproj
    h1 = pre * jax.nn.sigmoid(pre)
    z = jnp.dot(h1, wd_ref[...], preferred_element_type=jnp.float32) + bd_ref[...]
    gate = jax.nn.sigmoid(z[:, :EDGE_DIM])
    zo = z[:, EDGE_DIM:]
    outp = zo * jax.nn.sigmoid(zo)
    m_ref[...] = outp * gate * gnw_ref[:, 128:256]


def _final_body(p_ref, nw_ref, nf_ref, wl_ref, bl_ref, o_ref):
    h = (p_ref[0] + p_ref[1]) * nw_ref[...]
    o_ref[...] = (nf_ref[...]
                  + jnp.dot(h, wl_ref[...], preferred_element_type=jnp.float32)
                  + bl_ref[...])


_MESH = plsc.VectorSubcoreMesh(core_axis_name="c", subcore_axis_name="s",
                               num_cores=NC, num_subcores=NS)

_GSCRATCH = []
for _ in range(2):
    _GSCRATCH += [
        pltpu.VMEM((GCHUNK,), jnp.int32),
        pltpu.VMEM((GCHUNK,), jnp.int32),
        pltpu.VMEM((GCHUNK,), jnp.int32),
        pltpu.VMEM((GCHUNK, 256), jnp.float32),
        pltpu.VMEM((GCHUNK, 128), jnp.float32),
        pltpu.VMEM((GCHUNK, 128), jnp.float32),
        pltpu.SemaphoreType.DMA,
    ]


@functools.partial(
    pl.kernel,
    out_type=jax.ShapeDtypeStruct((E_PAD, 256), jnp.float32),
    mesh=_MESH,
    scratch_types=_GSCRATCH,
)
def _sc_gather(src_h, dst_h, vid_h, tsrc_h, tdst_h, tvert_h, gnw_out,
               sidx0, didx0, vidx0, snw0, drows0, vrows0, sem0,
               sidx1, didx1, vidx1, snw1, drows1, vrows1, sem1):
    c = lax.axis_index("c")
    s = lax.axis_index("s")
    base = (c * NS + s) * EPW
    sets = ((sidx0, didx0, vidx0, snw0, drows0, vrows0, sem0),
            (sidx1, didx1, vidx1, snw1, drows1, vrows1, sem1))

    def fire(bufs, cb):
        sidx, didx, vidx, snw, drows, vrows, sem = bufs
        pltpu.sync_copy(src_h.at[pl.ds(cb, GCHUNK)], sidx)
        pltpu.sync_copy(dst_h.at[pl.ds(cb, GCHUNK)], didx)
        pltpu.sync_copy(vid_h.at[pl.ds(cb, GCHUNK)], vidx)
        pltpu.async_copy(tsrc_h.at[sidx], snw, sem)
        pltpu.async_copy(tdst_h.at[didx], drows, sem)
        pltpu.async_copy(tvert_h.at[vidx], vrows, sem)

    def wait(bufs):
        sidx, didx, vidx, snw, drows, vrows, sem = bufs
        pltpu.make_async_copy(tsrc_h.at[sidx], snw, sem).wait()
        pltpu.make_async_copy(tdst_h.at[didx], drows, sem).wait()
        pltpu.make_async_copy(tvert_h.at[vidx], vrows, sem).wait()

    def consume(bufs, cb):
        sidx, didx, vidx, snw, drows, vrows, sem = bufs

        def edge_body(e, cin):
            for k in range(8):
                sl = pl.ds(k * 16, 16)
                snw[e, sl] = snw[e, sl] + drows[e, sl] + vrows[e, sl]
            return cin

        lax.fori_loop(0, GCHUNK, edge_body, 0)
        pltpu.sync_copy(snw, gnw_out.at[pl.ds(cb, GCHUNK)])

    fire(sets[0], base)

    def step(g, carry):
        cb = base + 2 * g * GCHUNK
        wait(sets[0])
        fire(sets[1], cb + GCHUNK)
        consume(sets[0], cb)
        wait(sets[1])
        fire(sets[0], cb + 2 * GCHUNK)
        consume(sets[1], cb + GCHUNK)
        return carry

    lax.fori_loop(0, GPAIRS, step, 0)
    wait(sets[0])


_SSCRATCH = []
for _ in range(2):
    _SSCRATCH += [
        pltpu.VMEM((SCHUNK,), jnp.int32),
        pltpu.VMEM((SCHUNK, 128), jnp.float32),
        pltpu.SemaphoreType.DMA,
    ]
_SSCRATCH += [
    pltpu.VMEM((STAIL,), jnp.int32),
    pltpu.VMEM((STAIL, 128), jnp.float32),
    pltpu.VMEM((ZROWS, 128), jnp.float32),
    pltpu.VMEM_SHARED((NB_PAD, 128), jnp.float32),
]


@functools.partial(
    pl.kernel,
    out_type=jax.ShapeDtypeStruct((NC, NB_PAD, 128), jnp.float32),
    mesh=_MESH,
    scratch_types=_SSCRATCH,
)
def _sc_scatter(m_h, dst_h, part_out,
                didx0, mrows0, sem0, didx1, mrows1, sem1,
                tidx, trows, zbuf, acc):
    c = lax.axis_index("c")
    s = lax.axis_index("s")
    sets = ((didx0, mrows0, sem0), (didx1, mrows1, sem1))

    def zrow(e, carry):
        for k in range(8):
            zbuf[e, pl.ds(k * 16, 16)] = jnp.zeros((16,), jnp.float32)
        return carry

    lax.fori_loop(0, ZROWS, zrow, 0)
    for j in range(RPT // ZROWS):
        pltpu.sync_copy(zbuf, acc.at[pl.ds(s * RPT + j * ZROWS, ZROWS)])
    plsc.subcore_barrier()

    base = (c * NS + s) * SPW

    def fire(bufs, cb):
        didx, mrows, sem = bufs
        pltpu.sync_copy(dst_h.at[pl.ds(cb, SCHUNK)], didx)
        pltpu.async_copy(m_h.at[pl.ds(cb, SCHUNK)], mrows, sem)

    def consume(bufs):
        didx, mrows, sem = bufs
        pltpu.make_async_copy(m_h.at[pl.ds(0, SCHUNK)], mrows, sem).wait()
        pltpu.sync_copy(mrows, acc.at[didx], add=True)

    fire(sets[0], base)

    def step(g, carry):
        cb = base + 2 * g * SCHUNK
        fire(sets[1], cb + SCHUNK)
        consume(sets[0])
        fire(sets[0], cb + 2 * SCHUNK)
        consume(sets[1])
        return carry

    lax.fori_loop(0, SPAIRS, step, 0)
    consume(sets[0])

    tb = base + SITER * SCHUNK
    pltpu.sync_copy(dst_h.at[pl.ds(tb, STAIL)], tidx)
    pltpu.sync_copy(m_h.at[pl.ds(tb, STAIL)], trows)
    pltpu.sync_copy(trows, acc.at[tidx], add=True)
    plsc.subcore_barrier()
    pltpu.sync_copy(acc.at[pl.ds(s * RPT, RPT)], part_out.at[c, pl.ds(s * RPT, RPT)])


def kernel(graph_edges, node_feat, edge_feat, node_weight, edge_index, atom_feat,
           gw_W1, gw_b1, gw_W2, gw_b2, out_W1, out_b1, out_W2, out_b2, lin_W, lin_b):
    f32 = jnp.float32
    src = graph_edges[0].astype(jnp.int32)
    dst = graph_edges[1].astype(jnp.int32)
    vid = edge_index[:, 1].astype(jnp.int32)
    pad = IDX_PAD - N_E
    src_p = jnp.pad(src, (0, pad))
    dst_p = jnp.pad(dst, (0, pad))
    vid_p = jnp.pad(vid, (0, pad))

    w_src = jnp.concatenate([gw_W1[0:128], out_W1[0:128]], axis=1)
    w_dst = jnp.concatenate([gw_W1[128:256], out_W1[128:256]], axis=1)
    w_edge = jnp.concatenate([gw_W1[256:384], out_W1[256:384]], axis=1)
    w_vert = jnp.concatenate([gw_W1[384:448], out_W1[384:448]], axis=1)
    b_edge = jnp.concatenate([gw_b1, out_b1]).reshape(1, 128)
    w_diag = jnp.zeros((128, 256), f32)
    w_diag = w_diag.at[0:64, 0:128].set(gw_W2).at[64:128, 128:256].set(out_W2)
    b_diag = jnp.concatenate([gw_b2, out_b2]).reshape(1, 256)
    b_lin = lin_b.reshape(1, 128)

    nb_blk = 2000
    t_srcnw, t_dst = pl.pallas_call(
        _node_tables_body,
        grid=(N_B // nb_blk,),
        in_specs=[pl.BlockSpec((nb_blk, 128), lambda i: (i, 0)),
                  pl.BlockSpec((nb_blk, 128), lambda i: (i, 0)),
                  pl.BlockSpec((128, 128), lambda i: (0, 0)),
                  pl.BlockSpec((128, 128), lambda i: (0, 0))],
        out_specs=[pl.BlockSpec((nb_blk, 256), lambda i: (i, 0)),
                   pl.BlockSpec((nb_blk, 128), lambda i: (i, 0))],
        out_shape=[jax.ShapeDtypeStruct((N_B, 256), f32),
                   jax.ShapeDtypeStruct((N_B, 128), f32)],
    )(node_feat, node_weight, w_src, w_dst)

    t_vert = pl.pallas_call(
        _vert_table_body,
        grid=(1,),
        in_specs=[pl.BlockSpec((N_A, ATOM_DIM), lambda i: (0, 0)),
                  pl.BlockSpec((ATOM_DIM, 128), lambda i: (0, 0))],
        out_specs=pl.BlockSpec((N_A, 128), lambda i: (0, 0)),
        out_shape=jax.ShapeDtypeStruct((N_A, 128), f32),
    )(atom_feat, w_vert)

    gnw = _sc_gather(src_p, dst_p, vid_p, t_srcnw, t_dst, t_vert)

    ne_blk = 2000
    m = pl.pallas_call(
        _edge_mlp_body,
        grid=(N_E // ne_blk,),
        in_specs=[pl.BlockSpec((ne_blk, 256), lambda i: (i, 0)),
                  pl.BlockSpec((ne_blk, 128), lambda i: (i, 0)),
                  pl.BlockSpec((128, 128), lambda i: (0, 0)),
                  pl.BlockSpec((1, 128), lambda i: (0, 0)),
                  pl.BlockSpec((128, 256), lambda i: (0, 0)),
                  pl.BlockSpec((1, 256), lambda i: (0, 0))],
        out_specs=pl.BlockSpec((ne_blk, 128), lambda i: (i, 0)),
        out_shape=jax.ShapeDtypeStruct((N_E, 128), f32),
    )(gnw, edge_feat, w_edge, b_edge, w_diag, b_diag)

    partials = _sc_scatter(m, dst)

    out = pl.pallas_call(
        _final_body,
        grid=(N_B // nb_blk,),
        in_specs=[pl.BlockSpec((NC, nb_blk, 128), lambda i: (0, i, 0)),
                  pl.BlockSpec((nb_blk, 128), lambda i: (i, 0)),
                  pl.BlockSpec((nb_blk, 128), lambda i: (i, 0)),
                  pl.BlockSpec((128, 128), lambda i: (0, 0)),
                  pl.BlockSpec((1, 128), lambda i: (0, 0))],
        out_specs=pl.BlockSpec((nb_blk, 128), lambda i: (i, 0)),
        out_shape=jax.ShapeDtypeStruct((N_B, 128), f32),
    )(partials, node_weight, node_feat, lin_W, b_lin)

    return out
